# baseline (speedup 1.0000x reference)
import dataclasses
import functools

import jax
import jax.numpy as jnp
from jax import lax
from jax.experimental import pallas as pl
from jax.experimental.pallas import tpu as pltpu
from jax.experimental.pallas import tpu_sc as plsc

def _sc_compiler_params(tc_tiling=True):
    cp = pltpu.CompilerParams()
    fields = pltpu.CompilerParams.__dataclass_fields__
    if "needs_layout_passes" in fields:
        cp = dataclasses.replace(cp, needs_layout_passes=False)
    if not tc_tiling and "use_tc_tiling_on_sc" in fields:
        cp = dataclasses.replace(cp, use_tc_tiling_on_sc=False)
    return cp


_NC = 2
_NS = 16
_NW = _NC * _NS
_LANES = 16
_BLK = 128


def _sc_degree(dst2d, w2d, n):
    rpt = dst2d.shape[0] // _NW
    zc = (n // _NS) // 8 * 8
    ztail = n - _NS * zc
    mesh = plsc.VectorSubcoreMesh(core_axis_name="c", subcore_axis_name="s")

    @functools.partial(
        pl.kernel,
        out_type=jax.ShapeDtypeStruct((_NC * n,), jnp.float32),
        mesh=mesh,
        scratch_types=[
            pltpu.VMEM((rpt, _BLK), jnp.int32),
            pltpu.VMEM((rpt, _BLK), jnp.float32),
            pltpu.VMEM((zc,), jnp.float32),
            pltpu.VMEM_SHARED((n,), jnp.float32),
        ],
    )
    def deg_kernel(dst_hbm, w_hbm, out_hbm, idx_v, w_v, buf_v, deg_sh):
        c = lax.axis_index("c")
        s = lax.axis_index("s")
        wid = c * _NS + s
        z0 = s * zc

        @pl.loop(0, zc // _LANES)
        def _(k):
            buf_v[pl.ds(k * _LANES, _LANES)] = jnp.zeros((_LANES,), jnp.float32)

        pltpu.sync_copy(buf_v, deg_sh.at[pl.ds(z0, zc)])

        @pl.when(s == _NS - 1)
        def _():
            if ztail:
                pltpu.sync_copy(buf_v.at[pl.ds(0, ztail)],
                                deg_sh.at[pl.ds(_NS * zc, ztail)])

        plsc.subcore_barrier()

        base = wid * rpt
        pltpu.sync_copy(dst_hbm.at[pl.ds(base, rpt)], idx_v)
        pltpu.sync_copy(w_hbm.at[pl.ds(base, rpt)], w_v)

        @pl.loop(0, rpt)
        def _(j):
            pltpu.sync_copy(w_v.at[j], deg_sh.at[idx_v.at[j]], add=True)

        plsc.subcore_barrier()
        pltpu.sync_copy(deg_sh.at[pl.ds(z0, zc)], buf_v)
        pltpu.sync_copy(buf_v, out_hbm.at[pl.ds(c * n + z0, zc)])

        @pl.when(s == _NS - 1)
        def _():
            if ztail:
                pltpu.sync_copy(deg_sh.at[pl.ds(_NS * zc, ztail)],
                                buf_v.at[pl.ds(0, ztail)])
                pltpu.sync_copy(buf_v.at[pl.ds(0, ztail)],
                                out_hbm.at[pl.ds(c * n + _NS * zc, ztail)])

    return deg_kernel(dst2d, w2d)


def _sc_messages(hd, src1d, dst1d, w1d):
    n = hd.shape[0]
    d = hd.shape[1] * 2
    rpt = src1d.shape[0] // (_NW * _BLK)
    zc = (n // _NS) // 8 * 8
    ztail = n - _NS * zc
    chunks = [(o, min(_BLK, zc - o)) for o in range(0, zc, _BLK)]
    mesh = plsc.VectorSubcoreMesh(core_axis_name="c", subcore_axis_name="s")

    step = 4
    assert rpt % step == 0

    scratch = [
        pltpu.VMEM((2, _BLK), jnp.int32),
        pltpu.VMEM((4, _BLK), jnp.int32),
        pltpu.VMEM((4, _BLK), jnp.float32),
    ]
    scratch += [pltpu.VMEM((_BLK, 64), jnp.int32) for _ in range(2)]
    scratch += [pltpu.VMEM((_BLK, 128), jnp.float32) for _ in range(2)]
    scratch += [pltpu.SemaphoreType.DMA for _ in range(2 + 2 + 2 + 4 + 4)]
    scratch += [pltpu.VMEM_SHARED((n, 128), jnp.float32)]

    @functools.partial(
        pl.kernel,
        out_type=jax.ShapeDtypeStruct((_NC, n, d), jnp.float32),
        mesh=mesh,
        compiler_params=_sc_compiler_params(tc_tiling=False),
        scratch_types=scratch,
    )
    def msg_kernel(hd_hbm, src_hbm, dst_hbm, w_hbm, out_hbm,
                   srcw, dstw, ww, *rest):
        gbufs = rest[0:2]
        fbufs = rest[2:4]
        gsem = rest[4:6]
        ssem = rest[6:8]
        sisem = rest[8:10]
        disem = rest[10:14]
        wsem = rest[14:18]
        acc_sh = rest[18]
        c = lax.axis_index("c")
        s = lax.axis_index("s")
        wid = c * _NS + s
        r0 = s * zc
        base = wid * rpt

        def src_dma(j, p):
            return pltpu.make_async_copy(
                src_hbm.at[pl.ds((base + j) * _BLK, _BLK)],
                srcw.at[p], sisem[p])

        def dst_dma(j, p):
            return pltpu.make_async_copy(
                dst_hbm.at[pl.ds((base + j) * _BLK, _BLK)],
                dstw.at[p], disem[p])

        def w_dma(j, p):
            return pltpu.make_async_copy(
                w_hbm.at[pl.ds((base + j) * _BLK, _BLK)],
                ww.at[p], wsem[p])

        def gather_start(b, p):
            pltpu.async_copy(hd_hbm.at[srcw.at[p]], gbufs[b], gsem[b])

        def gather_wait(b, p):
            pltpu.make_async_copy(hd_hbm.at[srcw.at[p]], gbufs[b],
                                  gsem[b]).wait()

        def scatter_start(b, p):
            pltpu.async_copy(fbufs[b], acc_sh.at[dstw.at[p]], ssem[b],
                             add=True)

        def scatter_wait(b, p):
            pltpu.make_async_copy(fbufs[b], acc_sh.at[dstw.at[p]],
                                  ssem[b]).wait()

        @pl.loop(0, _BLK)
        def _(i):
            for q in range(128 // _LANES):
                fbufs[0][i, pl.ds(q * _LANES, _LANES)] = (
                    jnp.zeros((_LANES,), jnp.float32))

        for off, nr in chunks:
            pltpu.sync_copy(fbufs[0].at[pl.ds(0, nr)],
                            acc_sh.at[pl.ds(r0 + off, nr)])

        @pl.when(s == _NS - 1)
        def _():
            if ztail:
                pltpu.sync_copy(fbufs[0].at[pl.ds(0, ztail)],
                                acc_sh.at[pl.ds(_NS * zc, ztail)])

        plsc.subcore_barrier()

        def scale_rows(gb, fb, wp):
            @pl.loop(0, _BLK, step=_LANES)
            def _(i0):
                wrow = ww[wp, pl.ds(i0, _LANES)]
                for u2 in range(_LANES):
                    wv = jnp.full((_LANES,), wrow[u2])
                    i = i0 + u2
                    for q in range(128 // 32):
                        ab = plsc.bitcast(gbufs[gb][i, pl.ds(q * 16, 16)],
                                          jnp.bfloat16)
                        lo, hi = plsc.unpack(
                            ab, format=plsc.PackFormat.INTERLEAVED)
                        fbufs[fb][i, pl.ds(q * 32, _LANES)] = lo * wv
                        fbufs[fb][i, pl.ds(q * 32 + 16, _LANES)] = hi * wv

        pltpu.sync_copy(src_hbm.at[pl.ds(base * _BLK, _BLK)], srcw.at[0])
        gather_start(0, 0)
        src_dma(1, 1).start()
        dst_dma(0, 0).start()
        w_dma(0, 0).start()

        @pl.loop(0, rpt, step=step)
        def _(j):
            for u in range(step):
                m = j + u
                gb = u % 2
                gather_wait(gb, u % 2)

                @pl.when(m + 2 < rpt)
                def _():
                    src_dma(m + 2, u % 2).start()

                @pl.when(m + 1 < rpt)
                def _():
                    dst_dma(m + 1, (u + 1) % 4).start()
                    w_dma(m + 1, (u + 1) % 4).start()
                    src_dma(m + 1, (u + 1) % 2).wait()
                    gather_start((u + 1) % 2, (u + 1) % 2)

                @pl.when(m - 2 >= 0)
                def _():
                    scatter_wait(u % 2, (u + 2) % 4)

                w_dma(m, u % 4).wait()
                scale_rows(gb, u % 2, u % 4)

                dst_dma(m, u % 4).wait()
                scatter_start(u % 2, u % 4)

        scatter_wait((rpt - 2) % 2, (rpt - 2) % 4)
        scatter_wait((rpt - 1) % 2, (rpt - 1) % 4)

        plsc.subcore_barrier()
        for off, nr in chunks:
            pltpu.sync_copy(acc_sh.at[pl.ds(r0 + off, nr)],
                            fbufs[0].at[pl.ds(0, nr)])
            pltpu.sync_copy(fbufs[0].at[pl.ds(0, nr)],
                            out_hbm.at[c, pl.ds(r0 + off, nr)])

        @pl.when(s == _NS - 1)
        def _():
            if ztail:
                pltpu.sync_copy(acc_sh.at[pl.ds(_NS * zc, ztail)],
                                fbufs[0].at[pl.ds(0, ztail)])
                pltpu.sync_copy(fbufs[0].at[pl.ds(0, ztail)],
                                out_hbm.at[c, pl.ds(_NS * zc, ztail)])

    return msg_kernel(hd, src1d, dst1d, w1d)


def _tc_matmul(x, W):
    n, d = x.shape
    blk = 1000

    def body(x_ref, w_ref, o_ref):
        o_ref[...] = jnp.dot(x_ref[...], w_ref[...],
                             preferred_element_type=jnp.float32)

    return pl.pallas_call(
        body,
        grid=(n // blk,),
        in_specs=[pl.BlockSpec((blk, d), lambda i: (i, 0)),
                  pl.BlockSpec((d, d), lambda i: (0, 0))],
        out_specs=pl.BlockSpec((blk, d), lambda i: (i, 0)),
        out_shape=jax.ShapeDtypeStruct((n, d), jnp.float32),
    )(x, W)


def _tc_scale(h, degT):
    n, d = h.shape
    blk = 1000

    def body(h_ref, g_ref, o_ref):
        dg = g_ref[:, 0:1] + g_ref[:, 1:2] + 1.0
        dinv = jnp.where(dg > 0, lax.rsqrt(dg), 0.0)
        val = h_ref[...] * dinv
        words = []
        for g in range(d // 32):
            lo = val[:, g * 32:g * 32 + 16].astype(jnp.bfloat16)
            hi = val[:, g * 32 + 16:g * 32 + 32].astype(jnp.bfloat16)
            lo32 = lax.bitcast_convert_type(lo, jnp.uint16).astype(jnp.uint32)
            hi32 = lax.bitcast_convert_type(hi, jnp.uint16).astype(jnp.uint32)
            words.append(lo32 | (hi32 << 16))
        o_ref[...] = lax.bitcast_convert_type(
            jnp.concatenate(words, axis=1), jnp.int32)

    return pl.pallas_call(
        body,
        grid=(n // blk,),
        in_specs=[pl.BlockSpec((blk, d), lambda i: (i, 0)),
                  pl.BlockSpec((blk, 2), lambda i: (i, 0))],
        out_specs=pl.BlockSpec((blk, d // 2), lambda i: (i, 0)),
        out_shape=jax.ShapeDtypeStruct((n, d // 2), jnp.int32),
    )(h, degT)


def _tc_combine(A2, h, degT, b):
    n, d = h.shape
    blk = 1000

    def body(a_ref, h_ref, g_ref, b_ref, o_ref):
        dg = g_ref[:, 0:1] + g_ref[:, 1:2] + 1.0
        dinv = jnp.where(dg > 0, lax.rsqrt(dg), 0.0)
        agg = (a_ref[0] + a_ref[1]) * dinv
        o_ref[...] = agg + h_ref[...] * (dinv * dinv) + b_ref[...]

    return pl.pallas_call(
        body,
        grid=(n // blk,),
        in_specs=[pl.BlockSpec((2, blk, d), lambda i: (0, i, 0)),
                  pl.BlockSpec((blk, d), lambda i: (i, 0)),
                  pl.BlockSpec((blk, 2), lambda i: (i, 0)),
                  pl.BlockSpec((1, d), lambda i: (0, 0))],
        out_specs=pl.BlockSpec((blk, d), lambda i: (i, 0)),
        out_shape=jax.ShapeDtypeStruct((n, d), jnp.float32),
    )(A2, h, degT, b)


def kernel(x, edge_index, edge_attr, W, b):
    n, d = x.shape
    e = edge_attr.shape[0]
    ei_flat = edge_index.reshape(-1)
    src = ei_flat[:e]
    dst = ei_flat[e:]
    w = edge_attr

    def pad_edges(rows):
        tgt = rows * _NW * _BLK
        padn = tgt - e
        if not padn:
            return src, dst, w
        fill = jnp.arange(padn, dtype=jnp.int32) % n
        return (jnp.concatenate([src, fill]),
                jnp.concatenate([dst, fill]),
                jnp.concatenate([w, jnp.zeros((padn,), jnp.float32)]))

    rpt0 = -(-e // (_NW * _BLK))
    srcm, dstm, wm = pad_edges(-(-rpt0 // 8) * 8)
    dstd, wd = dstm, wm

    deg2 = _sc_degree(dstd.reshape(-1, _BLK), wd.reshape(-1, _BLK), n)
    degT = deg2.reshape(2, n).T
    h = _tc_matmul(x, W)
    hd = _tc_scale(h, degT)
    A2 = _sc_messages(hd, srcm, dstm, wm)
    out2d = _tc_combine(A2, h, degT, b.reshape(1, d))

    seq = 8
    return jnp.transpose(out2d.reshape(n, seq, d // seq), (1, 0, 2))[None]
